# baseline (speedup 1.0000x reference)
import functools

import jax
import jax.numpy as jnp
from jax.experimental import pallas as pl
from jax.experimental.pallas import tpu as pltpu

_N = 20000
_G = 20
_NR = _N + _G
_ROWS = 160
_GROUPS = _ROWS // 8
_NP = _ROWS * 128
_K = 128
_KFG = 32
_STD = (0.1, 0.1, 0.2, 0.2)


def _body(coords_ref, gt_ref, nb_ref, info_ref, out_ref,
          prio_ref, s_ref, bg_ref):
    x1 = coords_ref[0, 0]
    y1 = coords_ref[0, 1]
    x2 = coords_ref[0, 2]
    y2 = coords_ref[0, 3]
    area = (x2 - x1 + 1.0) * (y2 - y1 + 1.0)

    nb = nb_ref[0, 0, 0]
    run_max = jnp.full((_ROWS, 128), -2.0, jnp.float32)
    best_g = jnp.zeros((_ROWS, 128), jnp.float32)
    for g in range(_G):
        gx1 = gt_ref[0, g, 0]
        gy1 = gt_ref[0, g, 1]
        gx2 = gt_ref[0, g, 2]
        gy2 = gt_ref[0, g, 3]
        iw = jnp.clip(jnp.minimum(x2, gx2) - jnp.maximum(x1, gx1) + 1.0, 0.0)
        ih = jnp.clip(jnp.minimum(y2, gy2) - jnp.maximum(y1, gy1) + 1.0, 0.0)
        inter = iw * ih
        garea = (gx2 - gx1 + 1.0) * (gy2 - gy1 + 1.0)
        iou = inter / (area + garea - inter + 1e-6)
        val = jnp.where(g < nb, iou, -1.0)
        upd = val > run_max
        run_max = jnp.where(upd, val, run_max)
        best_g = jnp.where(upd, jnp.float32(g), best_g)

    fg = run_max >= 0.5
    bgm = jnp.logical_and(run_max < 0.5, run_max >= 0.1)
    priority = run_max + jnp.where(fg, 10.0, 0.0) + jnp.where(bgm, 5.0, 0.0)

    gidx = (jax.lax.broadcasted_iota(jnp.int32, (_ROWS, 128), 0) * 128
            + jax.lax.broadcasted_iota(jnp.int32, (_ROWS, 128), 1)
            ).astype(jnp.float32)
    priority = jnp.where(gidx < float(_NR), priority, -1.0)

    prio_ref[...] = priority
    bg_ref[...] = best_g
    summary = jnp.max(priority.reshape(_GROUPS, 8, 128), axis=1)
    s_ref[...] = jnp.concatenate(
        [summary, jnp.full((24 - _GROUPS, 128), -5.0, jnp.float32)], axis=0)

    lane = jax.lax.broadcasted_iota(jnp.int32, (1, 128), 1).astype(jnp.float32)
    group_iota = jax.lax.broadcasted_iota(jnp.int32, (24, 128), 0)
    loc_idx = (jax.lax.broadcasted_iota(jnp.int32, (8, 128), 0) * 128
               + jax.lax.broadcasted_iota(jnp.int32, (8, 128), 1)
               ).astype(jnp.float32)

    def step(i, carry):
        sx1, sy1, sx2, sy2, sbg, sm = carry
        s = s_ref[...]
        m = jnp.max(s)
        g = jnp.min(jnp.where(s == m, group_iota, 10_000))
        start = g * 8
        block = prio_ref[pl.ds(start, 8), :]
        bidx = jnp.float32(128.0) * start.astype(jnp.float32) + loc_idx
        idx = jnp.min(jnp.where(block == m, bidx, 1e9))
        sel = bidx == idx
        newblock = jnp.where(sel, -3.0, block)
        prio_ref[pl.ds(start, 8), :] = newblock
        s_ref[pl.ds(g, 1), :] = jnp.max(newblock, axis=0, keepdims=True)
        selm = jnp.where(sel, 1.0, 0.0)
        onehot = jnp.where(lane == i.astype(jnp.float32), 1.0, 0.0)
        sx1 = sx1 + jnp.sum(selm * coords_ref[0, 0, pl.ds(start, 8), :]) * onehot
        sy1 = sy1 + jnp.sum(selm * coords_ref[0, 1, pl.ds(start, 8), :]) * onehot
        sx2 = sx2 + jnp.sum(selm * coords_ref[0, 2, pl.ds(start, 8), :]) * onehot
        sy2 = sy2 + jnp.sum(selm * coords_ref[0, 3, pl.ds(start, 8), :]) * onehot
        sbg = sbg + jnp.sum(selm * bg_ref[pl.ds(start, 8), :]) * onehot
        sm = sm + m * onehot
        return sx1, sy1, sx2, sy2, sbg, sm

    zero_row = jnp.zeros((1, 128), jnp.float32)
    sx1, sy1, sx2, sy2, sbg, sm = jax.lax.fori_loop(
        0, _K, step,
        (zero_row, zero_row, zero_row, zero_row, zero_row, zero_row))

    fg_row = sm >= 8.0
    sel_fg = jnp.logical_and(fg_row, lane < float(_KFG))

    lab = zero_row
    gx1r = zero_row
    gy1r = zero_row
    gx2r = zero_row
    gy2r = zero_row
    for g in range(_G):
        hit = sbg == jnp.float32(g)
        lab = jnp.where(hit, gt_ref[0, g, 4], lab)
        gx1r = jnp.where(hit, gt_ref[0, g, 0], gx1r)
        gy1r = jnp.where(hit, gt_ref[0, g, 1], gy1r)
        gx2r = jnp.where(hit, gt_ref[0, g, 2], gx2r)
        gy2r = jnp.where(hit, gt_ref[0, g, 3], gy2r)
    labels = jnp.where(sel_fg, lab, 0.0)

    ew = jnp.maximum(sx2 - sx1 + 1.0, 1e-6)
    eh = jnp.maximum(sy2 - sy1 + 1.0, 1e-6)
    ecx = sx1 + 0.5 * ew
    ecy = sy1 + 0.5 * eh
    gw = jnp.maximum(gx2r - gx1r + 1.0, 1e-6)
    gh = jnp.maximum(gy2r - gy1r + 1.0, 1e-6)
    gcx = gx1r + 0.5 * gw
    gcy = gy1r + 0.5 * gh
    dx = (gcx - ecx) / ew / _STD[0]
    dy = (gcy - ecy) / eh / _STD[1]
    dw = jnp.log(gw / ew) / _STD[2]
    dh = jnp.log(gh / eh) / _STD[3]
    fgf = jnp.where(sel_fg, 1.0, 0.0)
    dx = dx * fgf
    dy = dy * fgf
    dw = dw * fgf
    dh = dh * fgf

    infos = []
    for d in range(7):
        acc = zero_row
        for g in range(_G):
            acc = jnp.where(sbg == jnp.float32(g), info_ref[0, g, d], acc)
        infos.append(acc)

    rows = [sx1, sy1, sx2, sy2, labels, fgf, dx, dy, dw, dh,
            gx1r, gy1r, gx2r, gy2r] + infos + [zero_row, zero_row, zero_row]
    out_ref[0] = jnp.concatenate(rows, axis=0)


@jax.jit
def kernel(all_rois, gt_boxes, num_boxes, gt_3d_info):
    B = all_rois.shape[0]
    coords = jnp.concatenate([all_rois[:, :, 1:5], gt_boxes[:, :, :4]], axis=1)
    coords = jnp.pad(coords, ((0, 0), (0, _NP - _NR), (0, 0)))
    coords = coords.transpose(0, 2, 1).reshape(B, 4, _ROWS, 128)

    planes = pl.pallas_call(
        _body,
        grid=(B,),
        in_specs=[
            pl.BlockSpec((1, 4, _ROWS, 128), lambda b: (b, 0, 0, 0)),
            pl.BlockSpec((1, _G, 5), lambda b: (b, 0, 0),
                         memory_space=pltpu.SMEM),
            pl.BlockSpec((1, 1, 1), lambda b: (b, 0, 0),
                         memory_space=pltpu.SMEM),
            pl.BlockSpec((1, _G, 7), lambda b: (b, 0, 0),
                         memory_space=pltpu.SMEM),
        ],
        out_specs=pl.BlockSpec((1, 24, 128), lambda b: (b, 0, 0)),
        out_shape=jax.ShapeDtypeStruct((B, 24, 128), jnp.float32),
        scratch_shapes=[
            pltpu.VMEM((_ROWS, 128), jnp.float32),
            pltpu.VMEM((24, 128), jnp.float32),
            pltpu.VMEM((_ROWS, 128), jnp.float32),
        ],
    )(coords, gt_boxes, num_boxes.astype(jnp.int32).reshape(B, 1, 1),
      gt_3d_info)

    sx1 = planes[:, 0]
    sy1 = planes[:, 1]
    sx2 = planes[:, 2]
    sy2 = planes[:, 3]
    labels = planes[:, 4]
    fgf = planes[:, 5]
    rois = jnp.stack([jnp.zeros_like(sx1), sx1, sy1, sx2, sy2], axis=-1)
    bbox_targets = planes[:, 6:10].transpose(0, 2, 1)
    inside_w = jnp.broadcast_to(fgf[:, :, None], (B, _K, 4))
    outside_w = inside_w
    rois_for_3d = rois[:, :_KFG]
    gt_bbox_for_3d = planes[:, 10:14].transpose(0, 2, 1)[:, :_KFG]
    gt_3d_info_rois = planes[:, 14:21].transpose(0, 2, 1)[:, :_KFG]
    return (rois, labels, bbox_targets, inside_w, outside_w,
            rois_for_3d, gt_bbox_for_3d, gt_3d_info_rois)


# speedup vs baseline: 1.9934x; 1.9934x over previous
import functools

import jax
import jax.numpy as jnp
from jax.experimental import pallas as pl
from jax.experimental.pallas import tpu as pltpu

_N = 20000
_G = 20
_NR = _N + _G
_ROWS = 160
_NP = _ROWS * 128
_K = 128
_KFG = 32
_NCLS_STD = (0.1, 0.1, 0.2, 0.2)


def _body(coords_ref, gt_ref, nb_ref, info_ref, out_ref):
    x1 = coords_ref[0, 0]
    y1 = coords_ref[0, 1]
    x2 = coords_ref[0, 2]
    y2 = coords_ref[0, 3]
    area = (x2 - x1 + 1.0) * (y2 - y1 + 1.0)

    nb = nb_ref[0, 0, 0]
    run_max = jnp.full((_ROWS, 128), -2.0, jnp.float32)
    best_g = jnp.zeros((_ROWS, 128), jnp.float32)
    for g in range(_G):
        gx1 = gt_ref[0, g, 0]
        gy1 = gt_ref[0, g, 1]
        gx2 = gt_ref[0, g, 2]
        gy2 = gt_ref[0, g, 3]
        iw = jnp.clip(jnp.minimum(x2, gx2) - jnp.maximum(x1, gx1) + 1.0, 0.0)
        ih = jnp.clip(jnp.minimum(y2, gy2) - jnp.maximum(y1, gy1) + 1.0, 0.0)
        inter = iw * ih
        garea = (gx2 - gx1 + 1.0) * (gy2 - gy1 + 1.0)
        iou = inter / (area + garea - inter + 1e-6)
        val = jnp.where(g < nb, iou, -1.0)
        upd = val > run_max
        run_max = jnp.where(upd, val, run_max)
        best_g = jnp.where(upd, jnp.float32(g), best_g)

    fg = run_max >= 0.5
    bg = jnp.logical_and(run_max < 0.5, run_max >= 0.1)
    priority = run_max + jnp.where(fg, 10.0, 0.0) + jnp.where(bg, 5.0, 0.0)

    gidx = (jax.lax.broadcasted_iota(jnp.int32, (_ROWS, 128), 0) * 128
            + jax.lax.broadcasted_iota(jnp.int32, (_ROWS, 128), 1)
            ).astype(jnp.float32)
    priority = jnp.where(gidx < float(_NR), priority, -1.0)

    lane = jax.lax.broadcasted_iota(jnp.int32, (1, 128), 1).astype(jnp.float32)

    def step(i, carry):
        prio, kr, kc, sm = carry
        m = jnp.max(prio)
        idx = jnp.min(jnp.where(prio == m, gidx, 1e9))
        prio = jnp.where(gidx == idx, -3.0, prio)
        r = jnp.floor(idx * (1.0 / 128.0))
        c = idx - 128.0 * r
        onehot = jnp.where(lane == i.astype(jnp.float32), 1.0, 0.0)
        kr = kr + r * onehot
        kc = kc + c * onehot
        sm = sm + m * onehot
        return prio, kr, kc, sm

    zero_row = jnp.zeros((1, 128), jnp.float32)
    prio, kr, kc, sm = jax.lax.fori_loop(
        0, _K, step, (priority, zero_row, zero_row, zero_row))

    e_c = jnp.where(
        jax.lax.broadcasted_iota(jnp.int32, (128, 128), 0).astype(jnp.float32)
        == kc, 1.0, 0.0)
    e_r = jnp.where(
        jax.lax.broadcasted_iota(jnp.int32, (_ROWS, 128), 0).astype(jnp.float32)
        == kr, 1.0, 0.0)

    def pick(q):
        y = jax.lax.dot(q, e_c, precision=jax.lax.Precision.HIGHEST)
        return jnp.sum(e_r * y, axis=0, keepdims=True)

    sx1 = pick(x1)
    sy1 = pick(y1)
    sx2 = pick(x2)
    sy2 = pick(y2)
    sbg = pick(best_g)

    fg_row = sm >= 8.0
    sel_fg = jnp.logical_and(fg_row, lane < float(_KFG))

    lab = zero_row
    gx1r = zero_row
    gy1r = zero_row
    gx2r = zero_row
    gy2r = zero_row
    for g in range(_G):
        hit = sbg == jnp.float32(g)
        lab = jnp.where(hit, gt_ref[0, g, 4], lab)
        gx1r = jnp.where(hit, gt_ref[0, g, 0], gx1r)
        gy1r = jnp.where(hit, gt_ref[0, g, 1], gy1r)
        gx2r = jnp.where(hit, gt_ref[0, g, 2], gx2r)
        gy2r = jnp.where(hit, gt_ref[0, g, 3], gy2r)
    labels = jnp.where(sel_fg, lab, 0.0)

    ew = jnp.maximum(sx2 - sx1 + 1.0, 1e-6)
    eh = jnp.maximum(sy2 - sy1 + 1.0, 1e-6)
    ecx = sx1 + 0.5 * ew
    ecy = sy1 + 0.5 * eh
    gw = jnp.maximum(gx2r - gx1r + 1.0, 1e-6)
    gh = jnp.maximum(gy2r - gy1r + 1.0, 1e-6)
    gcx = gx1r + 0.5 * gw
    gcy = gy1r + 0.5 * gh
    dx = (gcx - ecx) / ew / _NCLS_STD[0]
    dy = (gcy - ecy) / eh / _NCLS_STD[1]
    dw = jnp.log(gw / ew) / _NCLS_STD[2]
    dh = jnp.log(gh / eh) / _NCLS_STD[3]
    fgf = jnp.where(sel_fg, 1.0, 0.0)
    dx = dx * fgf
    dy = dy * fgf
    dw = dw * fgf
    dh = dh * fgf

    infos = []
    for d in range(7):
        acc = zero_row
        for g in range(_G):
            acc = jnp.where(sbg == jnp.float32(g), info_ref[0, g, d], acc)
        infos.append(acc)

    rows = [sx1, sy1, sx2, sy2, labels, fgf, dx, dy, dw, dh,
            gx1r, gy1r, gx2r, gy2r] + infos + [zero_row, zero_row, zero_row]
    out_ref[0] = jnp.concatenate(rows, axis=0)


@jax.jit
def kernel(all_rois, gt_boxes, num_boxes, gt_3d_info):
    B = all_rois.shape[0]
    coords = jnp.concatenate([all_rois[:, :, 1:5], gt_boxes[:, :, :4]], axis=1)
    coords = jnp.pad(coords, ((0, 0), (0, _NP - _NR), (0, 0)))
    coords = coords.transpose(0, 2, 1).reshape(B, 4, _ROWS, 128)

    planes = pl.pallas_call(
        _body,
        grid=(B,),
        in_specs=[
            pl.BlockSpec((1, 4, _ROWS, 128), lambda b: (b, 0, 0, 0)),
            pl.BlockSpec((1, _G, 5), lambda b: (b, 0, 0),
                         memory_space=pltpu.SMEM),
            pl.BlockSpec((1, 1, 1), lambda b: (b, 0, 0),
                         memory_space=pltpu.SMEM),
            pl.BlockSpec((1, _G, 7), lambda b: (b, 0, 0),
                         memory_space=pltpu.SMEM),
        ],
        out_specs=pl.BlockSpec((1, 24, 128), lambda b: (b, 0, 0)),
        out_shape=jax.ShapeDtypeStruct((B, 24, 128), jnp.float32),
    )(coords, gt_boxes, num_boxes.astype(jnp.int32).reshape(B, 1, 1),
      gt_3d_info)

    sx1 = planes[:, 0]
    sy1 = planes[:, 1]
    sx2 = planes[:, 2]
    sy2 = planes[:, 3]
    labels = planes[:, 4]
    fgf = planes[:, 5]
    rois = jnp.stack([jnp.zeros_like(sx1), sx1, sy1, sx2, sy2], axis=-1)
    bbox_targets = planes[:, 6:10].transpose(0, 2, 1)
    inside_w = jnp.broadcast_to(fgf[:, :, None], (B, _K, 4))
    outside_w = inside_w
    rois_for_3d = rois[:, :_KFG]
    gt_bbox_for_3d = planes[:, 10:14].transpose(0, 2, 1)[:, :_KFG]
    gt_3d_info_rois = planes[:, 14:21].transpose(0, 2, 1)[:, :_KFG]
    return (rois, labels, bbox_targets, inside_w, outside_w,
            rois_for_3d, gt_bbox_for_3d, gt_3d_info_rois)


# speedup vs baseline: 4.5392x; 2.2772x over previous
import functools

import jax
import jax.numpy as jnp
from jax.experimental import pallas as pl
from jax.experimental.pallas import tpu as pltpu

_N = 20000
_G = 20
_NR = _N + _G
_ROWS = 160
_NP = _ROWS * 128
_K = 128
_KFG = 32
_B = 4
_STD = (0.1, 0.1, 0.2, 0.2)


def _body(coords_ref, gt_ref, nb_ref, info_ref, out_ref):
    x1 = coords_ref[:, 0]
    y1 = coords_ref[:, 1]
    x2 = coords_ref[:, 2]
    y2 = coords_ref[:, 3]
    area = (x2 - x1 + 1.0) * (y2 - y1 + 1.0)

    gtv = gt_ref[...]
    nbv = nb_ref[...]
    infov = info_ref[...]

    run_max = jnp.full((_B, _ROWS, 128), -2.0, jnp.float32)
    best_g = jnp.zeros((_B, _ROWS, 128), jnp.float32)
    for g in range(_G):
        gx1 = gtv[:, 0:1, g:g + 1]
        gy1 = gtv[:, 1:2, g:g + 1]
        gx2 = gtv[:, 2:3, g:g + 1]
        gy2 = gtv[:, 3:4, g:g + 1]
        iw = jnp.clip(jnp.minimum(x2, gx2) - jnp.maximum(x1, gx1) + 1.0, 0.0)
        ih = jnp.clip(jnp.minimum(y2, gy2) - jnp.maximum(y1, gy1) + 1.0, 0.0)
        inter = iw * ih
        garea = (gx2 - gx1 + 1.0) * (gy2 - gy1 + 1.0)
        iou = inter / (area + garea - inter + 1e-6)
        val = jnp.where(jnp.float32(g) < nbv, iou, -1.0)
        upd = val > run_max
        run_max = jnp.where(upd, val, run_max)
        best_g = jnp.where(upd, jnp.float32(g), best_g)

    fg = run_max >= 0.5
    bgm = jnp.logical_and(run_max < 0.5, run_max >= 0.1)
    priority = run_max + jnp.where(fg, 10.0, 0.0) + jnp.where(bgm, 5.0, 0.0)

    gidx = (jax.lax.broadcasted_iota(jnp.int32, (_B, _ROWS, 128), 1) * 128
            + jax.lax.broadcasted_iota(jnp.int32, (_B, _ROWS, 128), 2)
            ).astype(jnp.float32)
    priority = jnp.where(gidx < float(_NR), priority, -1.0)

    lane = jax.lax.broadcasted_iota(
        jnp.int32, (1, 1, 128), 2).astype(jnp.float32)

    def step(i, carry):
        prio, kr, kc, sm = carry
        m = jnp.max(jnp.max(prio, axis=2, keepdims=True),
                    axis=1, keepdims=True)
        cand = jnp.where(prio == m, gidx, 1e9)
        idx = jnp.min(jnp.min(cand, axis=2, keepdims=True),
                      axis=1, keepdims=True)
        prio = jnp.where(gidx == idx, -3.0, prio)
        r = jnp.floor(idx * (1.0 / 128.0))
        c = idx - 128.0 * r
        onehot = jnp.where(lane == i.astype(jnp.float32), 1.0, 0.0)
        kr = kr + r * onehot
        kc = kc + c * onehot
        sm = sm + m * onehot
        return prio, kr, kc, sm

    zero_row = jnp.zeros((_B, 1, 128), jnp.float32)
    prio, kr, kc, sm = jax.lax.fori_loop(
        0, _K, step, (priority, zero_row, zero_row, zero_row))

    e_c = jnp.where(
        jax.lax.broadcasted_iota(
            jnp.int32, (_B, 128, 128), 1).astype(jnp.float32) == kc,
        1.0, 0.0)
    e_r = jnp.where(
        jax.lax.broadcasted_iota(
            jnp.int32, (_B, _ROWS, 128), 1).astype(jnp.float32) == kr,
        1.0, 0.0)

    def pick(q):
        y = jax.lax.dot_general(
            q, e_c, dimension_numbers=(((2,), (1,)), ((0,), (0,))),
            precision=jax.lax.Precision.HIGHEST)
        return jnp.sum(e_r * y, axis=1, keepdims=True)

    sx1 = pick(x1)
    sy1 = pick(y1)
    sx2 = pick(x2)
    sy2 = pick(y2)
    sbg = pick(best_g)

    fg_row = sm >= 8.0
    sel_fg = jnp.logical_and(fg_row, lane < float(_KFG))

    zero_row = jnp.zeros((_B, 1, 128), jnp.float32)
    lab = zero_row
    gx1r = zero_row
    gy1r = zero_row
    gx2r = zero_row
    gy2r = zero_row
    for g in range(_G):
        hit = sbg == jnp.float32(g)
        lab = jnp.where(hit, gtv[:, 4:5, g:g + 1], lab)
        gx1r = jnp.where(hit, gtv[:, 0:1, g:g + 1], gx1r)
        gy1r = jnp.where(hit, gtv[:, 1:2, g:g + 1], gy1r)
        gx2r = jnp.where(hit, gtv[:, 2:3, g:g + 1], gx2r)
        gy2r = jnp.where(hit, gtv[:, 3:4, g:g + 1], gy2r)
    labels = jnp.where(sel_fg, lab, 0.0)

    ew = jnp.maximum(sx2 - sx1 + 1.0, 1e-6)
    eh = jnp.maximum(sy2 - sy1 + 1.0, 1e-6)
    ecx = sx1 + 0.5 * ew
    ecy = sy1 + 0.5 * eh
    gw = jnp.maximum(gx2r - gx1r + 1.0, 1e-6)
    gh = jnp.maximum(gy2r - gy1r + 1.0, 1e-6)
    gcx = gx1r + 0.5 * gw
    gcy = gy1r + 0.5 * gh
    dx = (gcx - ecx) / ew / _STD[0]
    dy = (gcy - ecy) / eh / _STD[1]
    dw = jnp.log(gw / ew) / _STD[2]
    dh = jnp.log(gh / eh) / _STD[3]
    fgf = jnp.where(sel_fg, 1.0, 0.0)
    dx = dx * fgf
    dy = dy * fgf
    dw = dw * fgf
    dh = dh * fgf

    infos = []
    for d in range(7):
        acc = zero_row
        for g in range(_G):
            acc = jnp.where(sbg == jnp.float32(g), infov[:, d:d + 1, g:g + 1],
                            acc)
        infos.append(acc)

    rows = [sx1, sy1, sx2, sy2, labels, fgf, dx, dy, dw, dh,
            gx1r, gy1r, gx2r, gy2r] + infos + [zero_row, zero_row, zero_row]
    out_ref[...] = jnp.concatenate(rows, axis=1)


@jax.jit
def kernel(all_rois, gt_boxes, num_boxes, gt_3d_info):
    B = all_rois.shape[0]
    coords = jnp.concatenate([all_rois[:, :, 1:5], gt_boxes[:, :, :4]], axis=1)
    coords = jnp.pad(coords, ((0, 0), (0, _NP - _NR), (0, 0)))
    coords = coords.transpose(0, 2, 1).reshape(B, 4, _ROWS, 128)

    gtv = jnp.pad(gt_boxes.transpose(0, 2, 1), ((0, 0), (0, 3), (0, 108)))
    nbv = jnp.broadcast_to(
        num_boxes.astype(jnp.float32)[:, None, None], (B, 1, 128))
    infov = jnp.pad(gt_3d_info.transpose(0, 2, 1), ((0, 0), (0, 1), (0, 108)))

    planes = pl.pallas_call(
        _body,
        grid=(1,),
        in_specs=[
            pl.BlockSpec((B, 4, _ROWS, 128), lambda b: (0, 0, 0, 0)),
            pl.BlockSpec((B, 8, 128), lambda b: (0, 0, 0)),
            pl.BlockSpec((B, 1, 128), lambda b: (0, 0, 0)),
            pl.BlockSpec((B, 8, 128), lambda b: (0, 0, 0)),
        ],
        out_specs=pl.BlockSpec((B, 24, 128), lambda b: (0, 0, 0)),
        out_shape=jax.ShapeDtypeStruct((B, 24, 128), jnp.float32),
    )(coords, gtv, nbv, infov)

    sx1 = planes[:, 0]
    sy1 = planes[:, 1]
    sx2 = planes[:, 2]
    sy2 = planes[:, 3]
    labels = planes[:, 4]
    fgf = planes[:, 5]
    rois = jnp.stack([jnp.zeros_like(sx1), sx1, sy1, sx2, sy2], axis=-1)
    bbox_targets = planes[:, 6:10].transpose(0, 2, 1)
    inside_w = jnp.broadcast_to(fgf[:, :, None], (B, _K, 4))
    outside_w = inside_w
    rois_for_3d = rois[:, :_KFG]
    gt_bbox_for_3d = planes[:, 10:14].transpose(0, 2, 1)[:, :_KFG]
    gt_3d_info_rois = planes[:, 14:21].transpose(0, 2, 1)[:, :_KFG]
    return (rois, labels, bbox_targets, inside_w, outside_w,
            rois_for_3d, gt_bbox_for_3d, gt_3d_info_rois)


# speedup vs baseline: 5.2952x; 1.1665x over previous
import functools

import jax
import jax.numpy as jnp
from jax import lax
from jax.experimental import pallas as pl
from jax.experimental.pallas import tpu as pltpu
from jax.experimental.pallas import tpu_sc as plsc

_N = 20000
_G = 20
_NR = _N + _G
_ROWS = 160
_NP = _ROWS * 128
_NCH = _NP // 16
_NL2 = _NCH // 16
_K = 128
_KFG = 32
_B = 4
_STD = (0.1, 0.1, 0.2, 0.2)


def _tc1_body(coords_ref, gt_ref, nb_ref, prio_ref, bg_ref):
    x1 = coords_ref[:, 0]
    y1 = coords_ref[:, 1]
    x2 = coords_ref[:, 2]
    y2 = coords_ref[:, 3]
    area = (x2 - x1 + 1.0) * (y2 - y1 + 1.0)

    gtv = gt_ref[...]
    nbv = nb_ref[...]

    run_max = jnp.full((_B, _ROWS, 128), -2.0, jnp.float32)
    best_g = jnp.zeros((_B, _ROWS, 128), jnp.float32)
    for g in range(_G):
        gx1 = gtv[:, 0:1, g:g + 1]
        gy1 = gtv[:, 1:2, g:g + 1]
        gx2 = gtv[:, 2:3, g:g + 1]
        gy2 = gtv[:, 3:4, g:g + 1]
        iw = jnp.clip(jnp.minimum(x2, gx2) - jnp.maximum(x1, gx1) + 1.0, 0.0)
        ih = jnp.clip(jnp.minimum(y2, gy2) - jnp.maximum(y1, gy1) + 1.0, 0.0)
        inter = iw * ih
        garea = (gx2 - gx1 + 1.0) * (gy2 - gy1 + 1.0)
        iou = inter / (area + garea - inter + 1e-6)
        val = jnp.where(jnp.float32(g) < nbv, iou, -1.0)
        upd = val > run_max
        run_max = jnp.where(upd, val, run_max)
        best_g = jnp.where(upd, jnp.float32(g), best_g)

    fg = run_max >= 0.5
    bgm = jnp.logical_and(run_max < 0.5, run_max >= 0.1)
    priority = run_max + jnp.where(fg, 10.0, 0.0) + jnp.where(bgm, 5.0, 0.0)

    gidx = (jax.lax.broadcasted_iota(jnp.int32, (_B, _ROWS, 128), 1) * 128
            + jax.lax.broadcasted_iota(jnp.int32, (_B, _ROWS, 128), 2)
            ).astype(jnp.float32)
    priority = jnp.where(gidx < float(_NR), priority, -1.0)

    prio_ref[...] = priority
    bg_ref[...] = best_g


def _first(mask, lane):
    return jnp.min(jnp.where(mask, lane, 10_000))


def _sc_body(prio_hbm, out_hbm, prio_v, cmax_v, l2_v, keep_v, keepm_v):
    cid = lax.axis_index("c")
    sid = lax.axis_index("s")

    @pl.when(sid < 2)
    def _():
        b = cid * 2 + sid
        pltpu.sync_copy(prio_hbm.at[b], prio_v)
        lane = lax.broadcasted_iota(jnp.int32, (16,), 0)

        def build_cmax(k, _):
            acc = jnp.full((16,), -9.0, jnp.float32)
            for j in range(16):
                v = prio_v[pl.ds((k * 16 + j) * 16, 16)]
                acc = jnp.where(lane == j, jnp.max(v), acc)
            cmax_v[pl.ds(k * 16, 16)] = acc
            return 0
        lax.fori_loop(0, _NL2, build_cmax, 0)

        def build_l2(k, _):
            acc = jnp.full((16,), -9.0, jnp.float32)
            for j in range(16):
                v = cmax_v[pl.ds((k * 16 + j) * 16, 16)]
                acc = jnp.where(lane == j, jnp.max(v), acc)
            l2_v[pl.ds(k * 16, 16)] = acc
            return 0
        lax.fori_loop(0, _NL2 // 16, build_l2, 0)

        l3 = jnp.full((16,), -9.0, jnp.float32)
        for h in range(_NL2 // 16):
            v = l2_v[pl.ds(h * 16, 16)]
            l3 = jnp.where(lane == h, jnp.max(v), l3)

        def outer(o, l3):
            ki = jnp.zeros((16,), jnp.float32)
            km = jnp.zeros((16,), jnp.float32)
            for j in range(16):
                m = jnp.max(l3)
                h = _first(l3 == m, lane)
                l2v = l2_v[pl.ds(h * 16, 16)]
                s2 = h * 16 + _first(l2v == m, lane)
                cmv = cmax_v[pl.ds(s2 * 16, 16)]
                s3 = s2 * 16 + _first(cmv == m, lane)
                pv = prio_v[pl.ds(s3 * 16, 16)]
                s4 = _first(pv == m, lane)
                idx = s3 * 16 + s4
                pv = jnp.where(lane == s4, -3.0, pv)
                prio_v[pl.ds(s3 * 16, 16)] = pv
                cmv = jnp.where(lane == (s3 - s2 * 16), jnp.max(pv), cmv)
                cmax_v[pl.ds(s2 * 16, 16)] = cmv
                l2v = jnp.where(lane == (s2 - h * 16), jnp.max(cmv), l2v)
                l2_v[pl.ds(h * 16, 16)] = l2v
                l3 = jnp.where(lane == h, jnp.max(l2v), l3)
                ki = jnp.where(lane == j, idx.astype(jnp.float32), ki)
                km = jnp.where(lane == j, m, km)
            keep_v[pl.ds(o * 16, 16)] = ki
            keepm_v[pl.ds(o * 16, 16)] = km
            return l3
        lax.fori_loop(0, _K // 16, outer, l3)

        pltpu.sync_copy(keep_v, out_hbm.at[b, 0])
        pltpu.sync_copy(keepm_v, out_hbm.at[b, 1])


_sc_select = functools.partial(
    pl.kernel,
    out_type=jax.ShapeDtypeStruct((_B, 2, _K), jnp.float32),
    mesh=plsc.VectorSubcoreMesh(core_axis_name="c", subcore_axis_name="s"),
    scratch_types=[
        pltpu.VMEM((_NP,), jnp.float32),
        pltpu.VMEM((_NCH,), jnp.float32),
        pltpu.VMEM((_NL2,), jnp.float32),
        pltpu.VMEM((_K,), jnp.float32),
        pltpu.VMEM((_K,), jnp.float32),
    ],
    compiler_params=pltpu.CompilerParams(needs_layout_passes=False),
)(_sc_body)


def _tc2_body(coords_ref, bgp_ref, sel_ref, gt_ref, info_ref, out_ref):
    x1 = coords_ref[:, 0]
    y1 = coords_ref[:, 1]
    x2 = coords_ref[:, 2]
    y2 = coords_ref[:, 3]
    best_g = bgp_ref[...]
    gtv = gt_ref[...]
    infov = info_ref[...]

    idx = sel_ref[:, 0:1, :]
    sm = sel_ref[:, 1:2, :]
    kr = jnp.floor(idx * (1.0 / 128.0))
    kc = idx - 128.0 * kr

    lane = jax.lax.broadcasted_iota(
        jnp.int32, (1, 1, 128), 2).astype(jnp.float32)

    e_c = jnp.where(
        jax.lax.broadcasted_iota(
            jnp.int32, (_B, 128, 128), 1).astype(jnp.float32) == kc,
        1.0, 0.0)
    e_r = jnp.where(
        jax.lax.broadcasted_iota(
            jnp.int32, (_B, _ROWS, 128), 1).astype(jnp.float32) == kr,
        1.0, 0.0)

    def pick(q):
        y = jax.lax.dot_general(
            q, e_c, dimension_numbers=(((2,), (1,)), ((0,), (0,))),
            precision=jax.lax.Precision.HIGHEST)
        return jnp.sum(e_r * y, axis=1, keepdims=True)

    sx1 = pick(x1)
    sy1 = pick(y1)
    sx2 = pick(x2)
    sy2 = pick(y2)
    sbg = pick(best_g)

    fg_row = sm >= 8.0
    sel_fg = jnp.logical_and(fg_row, lane < float(_KFG))

    zero_row = jnp.zeros((_B, 1, 128), jnp.float32)
    lab = zero_row
    gx1r = zero_row
    gy1r = zero_row
    gx2r = zero_row
    gy2r = zero_row
    for g in range(_G):
        hit = sbg == jnp.float32(g)
        lab = jnp.where(hit, gtv[:, 4:5, g:g + 1], lab)
        gx1r = jnp.where(hit, gtv[:, 0:1, g:g + 1], gx1r)
        gy1r = jnp.where(hit, gtv[:, 1:2, g:g + 1], gy1r)
        gx2r = jnp.where(hit, gtv[:, 2:3, g:g + 1], gx2r)
        gy2r = jnp.where(hit, gtv[:, 3:4, g:g + 1], gy2r)
    labels = jnp.where(sel_fg, lab, 0.0)

    ew = jnp.maximum(sx2 - sx1 + 1.0, 1e-6)
    eh = jnp.maximum(sy2 - sy1 + 1.0, 1e-6)
    ecx = sx1 + 0.5 * ew
    ecy = sy1 + 0.5 * eh
    gw = jnp.maximum(gx2r - gx1r + 1.0, 1e-6)
    gh = jnp.maximum(gy2r - gy1r + 1.0, 1e-6)
    gcx = gx1r + 0.5 * gw
    gcy = gy1r + 0.5 * gh
    dx = (gcx - ecx) / ew / _STD[0]
    dy = (gcy - ecy) / eh / _STD[1]
    dw = jnp.log(gw / ew) / _STD[2]
    dh = jnp.log(gh / eh) / _STD[3]
    fgf = jnp.where(sel_fg, 1.0, 0.0)
    dx = dx * fgf
    dy = dy * fgf
    dw = dw * fgf
    dh = dh * fgf

    infos = []
    for d in range(7):
        acc = zero_row
        for g in range(_G):
            acc = jnp.where(sbg == jnp.float32(g), infov[:, d:d + 1, g:g + 1],
                            acc)
        infos.append(acc)

    rows = [sx1, sy1, sx2, sy2, labels, fgf, dx, dy, dw, dh,
            gx1r, gy1r, gx2r, gy2r] + infos + [zero_row, zero_row, zero_row]
    out_ref[...] = jnp.concatenate(rows, axis=1)


@jax.jit
def kernel(all_rois, gt_boxes, num_boxes, gt_3d_info):
    B = all_rois.shape[0]
    coords = jnp.concatenate([all_rois[:, :, 1:5], gt_boxes[:, :, :4]], axis=1)
    coords = jnp.pad(coords, ((0, 0), (0, _NP - _NR), (0, 0)))
    coords = coords.transpose(0, 2, 1).reshape(B, 4, _ROWS, 128)

    gtv = jnp.pad(gt_boxes.transpose(0, 2, 1), ((0, 0), (0, 3), (0, 108)))
    nbv = jnp.broadcast_to(
        num_boxes.astype(jnp.float32)[:, None, None], (B, 1, 128))
    infov = jnp.pad(gt_3d_info.transpose(0, 2, 1), ((0, 0), (0, 1), (0, 108)))

    prio, bgp = pl.pallas_call(
        _tc1_body,
        grid=(1,),
        in_specs=[
            pl.BlockSpec((B, 4, _ROWS, 128), lambda b: (0, 0, 0, 0)),
            pl.BlockSpec((B, 8, 128), lambda b: (0, 0, 0)),
            pl.BlockSpec((B, 1, 128), lambda b: (0, 0, 0)),
        ],
        out_specs=[
            pl.BlockSpec((B, _ROWS, 128), lambda b: (0, 0, 0)),
            pl.BlockSpec((B, _ROWS, 128), lambda b: (0, 0, 0)),
        ],
        out_shape=[
            jax.ShapeDtypeStruct((B, _ROWS, 128), jnp.float32),
            jax.ShapeDtypeStruct((B, _ROWS, 128), jnp.float32),
        ],
    )(coords, gtv, nbv)

    sel = _sc_select(prio.reshape(B, _NP))

    planes = pl.pallas_call(
        _tc2_body,
        grid=(1,),
        in_specs=[
            pl.BlockSpec((B, 4, _ROWS, 128), lambda b: (0, 0, 0, 0)),
            pl.BlockSpec((B, _ROWS, 128), lambda b: (0, 0, 0)),
            pl.BlockSpec((B, 2, _K), lambda b: (0, 0, 0)),
            pl.BlockSpec((B, 8, 128), lambda b: (0, 0, 0)),
            pl.BlockSpec((B, 8, 128), lambda b: (0, 0, 0)),
        ],
        out_specs=pl.BlockSpec((B, 24, 128), lambda b: (0, 0, 0)),
        out_shape=jax.ShapeDtypeStruct((B, 24, 128), jnp.float32),
    )(coords, bgp, sel, gtv, infov)

    sx1 = planes[:, 0]
    sy1 = planes[:, 1]
    sx2 = planes[:, 2]
    sy2 = planes[:, 3]
    labels = planes[:, 4]
    fgf = planes[:, 5]
    rois = jnp.stack([jnp.zeros_like(sx1), sx1, sy1, sx2, sy2], axis=-1)
    bbox_targets = planes[:, 6:10].transpose(0, 2, 1)
    inside_w = jnp.broadcast_to(fgf[:, :, None], (B, _K, 4))
    outside_w = inside_w
    rois_for_3d = rois[:, :_KFG]
    gt_bbox_for_3d = planes[:, 10:14].transpose(0, 2, 1)[:, :_KFG]
    gt_3d_info_rois = planes[:, 14:21].transpose(0, 2, 1)[:, :_KFG]
    return (rois, labels, bbox_targets, inside_w, outside_w,
            rois_for_3d, gt_bbox_for_3d, gt_3d_info_rois)


# speedup vs baseline: 5.3610x; 1.0124x over previous
import functools

import jax
import jax.numpy as jnp
from jax import lax
from jax.experimental import pallas as pl
from jax.experimental.pallas import tpu as pltpu
from jax.experimental.pallas import tpu_sc as plsc

_N = 20000
_G = 20
_NR = _N + _G
_ROWS = 160
_NP = _ROWS * 128
_NCH = _NP // 16
_NL2 = _NCH // 16
_K = 128
_KFG = 32
_B = 4
_STD = (0.1, 0.1, 0.2, 0.2)


def _tc1_body(coords_ref, gt_ref, nb_ref, prio_ref):
    x1 = coords_ref[:, 0]
    y1 = coords_ref[:, 1]
    x2 = coords_ref[:, 2]
    y2 = coords_ref[:, 3]
    area = (x2 - x1 + 1.0) * (y2 - y1 + 1.0)

    gtv = gt_ref[...]
    nbv = nb_ref[...]

    run_max = jnp.full((_B, _ROWS, 128), -2.0, jnp.float32)
    for g in range(_G):
        gx1 = gtv[:, 0:1, g:g + 1]
        gy1 = gtv[:, 1:2, g:g + 1]
        gx2 = gtv[:, 2:3, g:g + 1]
        gy2 = gtv[:, 3:4, g:g + 1]
        iw = jnp.clip(jnp.minimum(x2, gx2) - jnp.maximum(x1, gx1) + 1.0, 0.0)
        ih = jnp.clip(jnp.minimum(y2, gy2) - jnp.maximum(y1, gy1) + 1.0, 0.0)
        inter = iw * ih
        garea = (gx2 - gx1 + 1.0) * (gy2 - gy1 + 1.0)
        iou = inter / (area + garea - inter + 1e-6)
        val = jnp.where(jnp.float32(g) < nbv, iou, -1.0)
        run_max = jnp.maximum(run_max, val)

    fg = run_max >= 0.5
    bgm = jnp.logical_and(run_max < 0.5, run_max >= 0.1)
    priority = run_max + jnp.where(fg, 10.0, 0.0) + jnp.where(bgm, 5.0, 0.0)

    gidx = (jax.lax.broadcasted_iota(jnp.int32, (_B, _ROWS, 128), 1) * 128
            + jax.lax.broadcasted_iota(jnp.int32, (_B, _ROWS, 128), 2)
            ).astype(jnp.float32)
    priority = jnp.where(gidx < float(_NR), priority, -1.0)

    prio_ref[...] = priority


def _first(mask, lane):
    return jnp.min(jnp.where(mask, lane, 10_000))


def _sc_body(prio_hbm, out_hbm, prio_v, cmax_v, l2_v, keep_v, keepm_v):
    cid = lax.axis_index("c")
    sid = lax.axis_index("s")

    @pl.when(sid < 2)
    def _():
        b = cid * 2 + sid
        pltpu.sync_copy(prio_hbm.at[b], prio_v)
        lane = lax.broadcasted_iota(jnp.int32, (16,), 0)

        def build_cmax(k, _):
            acc = jnp.full((16,), -9.0, jnp.float32)
            for j in range(16):
                v = prio_v[pl.ds((k * 16 + j) * 16, 16)]
                acc = jnp.where(lane == j, jnp.max(v), acc)
            cmax_v[pl.ds(k * 16, 16)] = acc
            return 0
        lax.fori_loop(0, _NL2, build_cmax, 0)

        def build_l2(k, _):
            acc = jnp.full((16,), -9.0, jnp.float32)
            for j in range(16):
                v = cmax_v[pl.ds((k * 16 + j) * 16, 16)]
                acc = jnp.where(lane == j, jnp.max(v), acc)
            l2_v[pl.ds(k * 16, 16)] = acc
            return 0
        lax.fori_loop(0, _NL2 // 16, build_l2, 0)

        l3 = jnp.full((16,), -9.0, jnp.float32)
        for h in range(_NL2 // 16):
            v = l2_v[pl.ds(h * 16, 16)]
            l3 = jnp.where(lane == h, jnp.max(v), l3)

        def outer(o, l3):
            ki = jnp.zeros((16,), jnp.float32)
            km = jnp.zeros((16,), jnp.float32)
            for j in range(16):
                m = jnp.max(l3)
                h = _first(l3 == m, lane)
                l2v = l2_v[pl.ds(h * 16, 16)]
                s2 = h * 16 + _first(l2v == m, lane)
                cmv = cmax_v[pl.ds(s2 * 16, 16)]
                s3 = s2 * 16 + _first(cmv == m, lane)
                pv = prio_v[pl.ds(s3 * 16, 16)]
                s4 = _first(pv == m, lane)
                idx = s3 * 16 + s4
                pv = jnp.where(lane == s4, -3.0, pv)
                prio_v[pl.ds(s3 * 16, 16)] = pv
                cmv = jnp.where(lane == (s3 - s2 * 16), jnp.max(pv), cmv)
                cmax_v[pl.ds(s2 * 16, 16)] = cmv
                l2v = jnp.where(lane == (s2 - h * 16), jnp.max(cmv), l2v)
                l2_v[pl.ds(h * 16, 16)] = l2v
                l3 = jnp.where(lane == h, jnp.max(l2v), l3)
                ki = jnp.where(lane == j, idx.astype(jnp.float32), ki)
                km = jnp.where(lane == j, m, km)
            keep_v[pl.ds(o * 16, 16)] = ki
            keepm_v[pl.ds(o * 16, 16)] = km
            return l3
        lax.fori_loop(0, _K // 16, outer, l3)

        pltpu.sync_copy(keep_v, out_hbm.at[b, 0])
        pltpu.sync_copy(keepm_v, out_hbm.at[b, 1])


_sc_select = functools.partial(
    pl.kernel,
    out_type=jax.ShapeDtypeStruct((_B, 2, _K), jnp.float32),
    mesh=plsc.VectorSubcoreMesh(core_axis_name="c", subcore_axis_name="s"),
    scratch_types=[
        pltpu.VMEM((_NP,), jnp.float32),
        pltpu.VMEM((_NCH,), jnp.float32),
        pltpu.VMEM((_NL2,), jnp.float32),
        pltpu.VMEM((_K,), jnp.float32),
        pltpu.VMEM((_K,), jnp.float32),
    ],
    compiler_params=pltpu.CompilerParams(needs_layout_passes=False),
)(_sc_body)


def _tc2_body(coords_ref, nb_ref, sel_ref, gt_ref, info_ref, out_ref):
    x1 = coords_ref[:, 0]
    y1 = coords_ref[:, 1]
    x2 = coords_ref[:, 2]
    y2 = coords_ref[:, 3]
    nbv = nb_ref[...]
    gtv = gt_ref[...]
    infov = info_ref[...]

    idx = sel_ref[:, 0:1, :]
    sm = sel_ref[:, 1:2, :]
    kr = jnp.floor(idx * (1.0 / 128.0))
    kc = idx - 128.0 * kr

    lane = jax.lax.broadcasted_iota(
        jnp.int32, (1, 1, 128), 2).astype(jnp.float32)

    e_c = jnp.where(
        jax.lax.broadcasted_iota(
            jnp.int32, (_B, 128, 128), 1).astype(jnp.float32) == kc,
        1.0, 0.0)
    e_r = jnp.where(
        jax.lax.broadcasted_iota(
            jnp.int32, (_B, _ROWS, 128), 1).astype(jnp.float32) == kr,
        1.0, 0.0)

    def pick(q):
        y = jax.lax.dot_general(
            q, e_c, dimension_numbers=(((2,), (1,)), ((0,), (0,))),
            precision=jax.lax.Precision.HIGHEST)
        return jnp.sum(e_r * y, axis=1, keepdims=True)

    sx1 = pick(x1)
    sy1 = pick(y1)
    sx2 = pick(x2)
    sy2 = pick(y2)

    sarea = (sx2 - sx1 + 1.0) * (sy2 - sy1 + 1.0)
    run_max = jnp.full(sx1.shape, -2.0, jnp.float32)
    sbg = jnp.zeros(sx1.shape, jnp.float32)
    for g in range(_G):
        gx1 = gtv[:, 0:1, g:g + 1]
        gy1 = gtv[:, 1:2, g:g + 1]
        gx2 = gtv[:, 2:3, g:g + 1]
        gy2 = gtv[:, 3:4, g:g + 1]
        iw = jnp.clip(jnp.minimum(sx2, gx2) - jnp.maximum(sx1, gx1) + 1.0, 0.0)
        ih = jnp.clip(jnp.minimum(sy2, gy2) - jnp.maximum(sy1, gy1) + 1.0, 0.0)
        inter = iw * ih
        garea = (gx2 - gx1 + 1.0) * (gy2 - gy1 + 1.0)
        iou = inter / (sarea + garea - inter + 1e-6)
        val = jnp.where(jnp.float32(g) < nbv, iou, -1.0)
        upd = val > run_max
        run_max = jnp.where(upd, val, run_max)
        sbg = jnp.where(upd, jnp.float32(g), sbg)

    fg_row = sm >= 8.0
    sel_fg = jnp.logical_and(fg_row, lane < float(_KFG))

    zero_row = jnp.zeros((_B, 1, 128), jnp.float32)
    lab = zero_row
    gx1r = zero_row
    gy1r = zero_row
    gx2r = zero_row
    gy2r = zero_row
    for g in range(_G):
        hit = sbg == jnp.float32(g)
        lab = jnp.where(hit, gtv[:, 4:5, g:g + 1], lab)
        gx1r = jnp.where(hit, gtv[:, 0:1, g:g + 1], gx1r)
        gy1r = jnp.where(hit, gtv[:, 1:2, g:g + 1], gy1r)
        gx2r = jnp.where(hit, gtv[:, 2:3, g:g + 1], gx2r)
        gy2r = jnp.where(hit, gtv[:, 3:4, g:g + 1], gy2r)
    labels = jnp.where(sel_fg, lab, 0.0)

    ew = jnp.maximum(sx2 - sx1 + 1.0, 1e-6)
    eh = jnp.maximum(sy2 - sy1 + 1.0, 1e-6)
    ecx = sx1 + 0.5 * ew
    ecy = sy1 + 0.5 * eh
    gw = jnp.maximum(gx2r - gx1r + 1.0, 1e-6)
    gh = jnp.maximum(gy2r - gy1r + 1.0, 1e-6)
    gcx = gx1r + 0.5 * gw
    gcy = gy1r + 0.5 * gh
    dx = (gcx - ecx) / ew / _STD[0]
    dy = (gcy - ecy) / eh / _STD[1]
    dw = jnp.log(gw / ew) / _STD[2]
    dh = jnp.log(gh / eh) / _STD[3]
    fgf = jnp.where(sel_fg, 1.0, 0.0)
    dx = dx * fgf
    dy = dy * fgf
    dw = dw * fgf
    dh = dh * fgf

    infos = []
    for d in range(7):
        acc = zero_row
        for g in range(_G):
            acc = jnp.where(sbg == jnp.float32(g), infov[:, d:d + 1, g:g + 1],
                            acc)
        infos.append(acc)

    rows = [sx1, sy1, sx2, sy2, labels, fgf, dx, dy, dw, dh,
            gx1r, gy1r, gx2r, gy2r] + infos + [zero_row, zero_row, zero_row]
    out_ref[...] = jnp.concatenate(rows, axis=1)


@jax.jit
def kernel(all_rois, gt_boxes, num_boxes, gt_3d_info):
    B = all_rois.shape[0]
    coords = jnp.concatenate([all_rois[:, :, 1:5], gt_boxes[:, :, :4]], axis=1)
    coords = jnp.pad(coords, ((0, 0), (0, _NP - _NR), (0, 0)))
    coords = coords.transpose(0, 2, 1).reshape(B, 4, _ROWS, 128)

    gtv = jnp.pad(gt_boxes.transpose(0, 2, 1), ((0, 0), (0, 3), (0, 108)))
    nbv = jnp.broadcast_to(
        num_boxes.astype(jnp.float32)[:, None, None], (B, 1, 128))
    infov = jnp.pad(gt_3d_info.transpose(0, 2, 1), ((0, 0), (0, 1), (0, 108)))

    prio = pl.pallas_call(
        _tc1_body,
        grid=(1,),
        in_specs=[
            pl.BlockSpec((B, 4, _ROWS, 128), lambda b: (0, 0, 0, 0)),
            pl.BlockSpec((B, 8, 128), lambda b: (0, 0, 0)),
            pl.BlockSpec((B, 1, 128), lambda b: (0, 0, 0)),
        ],
        out_specs=pl.BlockSpec((B, _ROWS, 128), lambda b: (0, 0, 0)),
        out_shape=jax.ShapeDtypeStruct((B, _ROWS, 128), jnp.float32),
    )(coords, gtv, nbv)

    sel = _sc_select(prio.reshape(B, _NP))

    planes = pl.pallas_call(
        _tc2_body,
        grid=(1,),
        in_specs=[
            pl.BlockSpec((B, 4, _ROWS, 128), lambda b: (0, 0, 0, 0)),
            pl.BlockSpec((B, 1, 128), lambda b: (0, 0, 0)),
            pl.BlockSpec((B, 2, _K), lambda b: (0, 0, 0)),
            pl.BlockSpec((B, 8, 128), lambda b: (0, 0, 0)),
            pl.BlockSpec((B, 8, 128), lambda b: (0, 0, 0)),
        ],
        out_specs=pl.BlockSpec((B, 24, 128), lambda b: (0, 0, 0)),
        out_shape=jax.ShapeDtypeStruct((B, 24, 128), jnp.float32),
    )(coords, nbv, sel, gtv, infov)

    sx1 = planes[:, 0]
    sy1 = planes[:, 1]
    sx2 = planes[:, 2]
    sy2 = planes[:, 3]
    labels = planes[:, 4]
    fgf = planes[:, 5]
    rois = jnp.stack([jnp.zeros_like(sx1), sx1, sy1, sx2, sy2], axis=-1)
    bbox_targets = planes[:, 6:10].transpose(0, 2, 1)
    inside_w = jnp.broadcast_to(fgf[:, :, None], (B, _K, 4))
    outside_w = inside_w
    rois_for_3d = rois[:, :_KFG]
    gt_bbox_for_3d = planes[:, 10:14].transpose(0, 2, 1)[:, :_KFG]
    gt_3d_info_rois = planes[:, 14:21].transpose(0, 2, 1)[:, :_KFG]
    return (rois, labels, bbox_targets, inside_w, outside_w,
            rois_for_3d, gt_bbox_for_3d, gt_3d_info_rois)
